# baseline (speedup 1.0000x reference)
import jax
import jax.numpy as jnp
from jax import lax
from jax.experimental import pallas as pl
from jax.experimental.pallas import tpu as pltpu
from jax.experimental.pallas import tpu_sc as plsc

_BATCH = 4096
_SEQ = 200
_EMBED = 64
_LANES = 16
_NC = 2
_NS = 16
_NW = _NC * _NS
_BTILE = _BATCH // _NW
_STILE = _SEQ // 8
_NBUF = 4


def _sc_body(x4_hbm, tok_hbm, pos_hbm, out_hbm, idx_v, gath_v, outb_v,
             pos_v, *sems):
    semg = sems[:_NBUF]
    semo = sems[_NBUF:]
    wid = lax.axis_index("s") * _NC + lax.axis_index("c")
    pltpu.sync_copy(pos_hbm, pos_v)
    pltpu.sync_copy(x4_hbm.at[:, wid], idx_v)

    evecs = [lax.iota(jnp.int32, 16) + 16 * ec for ec in range(4)]
    ehis = [(lax.iota(jnp.int32, 16) + 16 * ec) // 8 for ec in range(4)]
    elos = [(lax.iota(jnp.int32, 16) + 16 * ec) % 8 for ec in range(4)]

    def fire_gather(s, b):
        pltpu.async_copy(
            tok_hbm.at[idx_v.at[s // 8, s % 8]], gath_v.at[b], semg[b])

    for b in range(_NBUF):
        fire_gather(b, b)

    def step_body(it, carry):
        for b in range(_NBUF):
            s = it * _NBUF + b
            pltpu.make_async_copy(
                tok_hbm.at[pl.ds(0, _BTILE)], gath_v.at[b], semg[b]).wait()
            pl.when(it > 0)(
                lambda: pltpu.make_async_copy(
                    tok_hbm.at[pl.ds(0, _BTILE)], gath_v.at[b],
                    semo[b]).wait())

            g = gath_v.at[b]
            bufv = jnp.full((16,), b, jnp.int32)
            pcs = [pos_v[s, pl.ds(ec * _LANES, _LANES)] for ec in range(4)]

            @plsc.parallel_loop(0, _BTILE, 1, unroll=32)
            def add_t(bb):
                bv = jnp.full((16,), bb, jnp.int32)
                for ec in range(4):
                    vals = g[bb, pl.ds(ec * _LANES, _LANES)] + pcs[ec]
                    plsc.store_scatter(outb_v, [bufv, ehis[ec], elos[ec], bv],
                                       vals)

            pl.when(it < (_SEQ // _NBUF) - 1)(lambda: fire_gather(s + _NBUF, b))
            pltpu.async_copy(
                outb_v.at[b, :, :, pl.ds(0, _BTILE)],
                out_hbm.at[s, pl.ds(0, 8), wid], semo[b])
        return carry

    lax.fori_loop(0, _SEQ // _NBUF, step_body, 0)
    for b in range(_NBUF):
        pltpu.make_async_copy(
            tok_hbm.at[pl.ds(0, _BTILE)], gath_v.at[b], semo[b]).wait()


def kernel(x, token_table, pos_table):
    x32 = x.astype(jnp.int32)
    x4 = x32.T.reshape(_STILE, 8, _NW, _BTILE).transpose(0, 2, 1, 3)
    f = pl.kernel(
        _sc_body,
        mesh=plsc.VectorSubcoreMesh(core_axis_name="c", subcore_axis_name="s"),
        compiler_params=pltpu.CompilerParams(
            use_tc_tiling_on_sc=False, needs_layout_passes=False),
        out_type=jax.ShapeDtypeStruct((_SEQ, 8, _NW, 8, _BTILE), jnp.float32),
        scratch_types=[
            pltpu.VMEM((_STILE, 8, _BTILE), jnp.int32),
            pltpu.VMEM((_NBUF, _BTILE, _EMBED), jnp.float32),
            pltpu.VMEM((_NBUF, 8, 8, _BTILE + 1), jnp.float32),
            pltpu.VMEM((_SEQ, _EMBED), jnp.float32),
        ] + [pltpu.SemaphoreType.DMA] * (2 * _NBUF),
    )
    y5 = f(x4, token_table, pos_table)
    return y5.transpose(2, 4, 0, 1, 3).reshape(_BATCH, _SEQ, _EMBED)


# speedup vs baseline: 1.4590x; 1.4590x over previous
import jax
import jax.numpy as jnp
from jax import lax
from jax.experimental import pallas as pl
from jax.experimental.pallas import tpu as pltpu
from jax.experimental.pallas import tpu_sc as plsc

_BATCH = 4096
_SEQ = 200
_EMBED = 64
_LANES = 16
_NC = 2
_NS = 16
_NW = _NC * _NS
_BTILE = _BATCH // _NW
_STILE = _SEQ // 8
_NBUF = 4


def _sc_body(x4_hbm, tok_hbm, pos_hbm, out_hbm, idx_v, gath_v, outb_v,
             pos_v, *sems):
    semg = sems[:_NBUF]
    semo = sems[_NBUF:2 * _NBUF]
    semi, semp = sems[2 * _NBUF], sems[2 * _NBUF + 1]
    wid = lax.axis_index("s") * _NC + lax.axis_index("c")
    idx_cp = pltpu.make_async_copy(x4_hbm.at[:, wid], idx_v, semi)
    pos_cp = pltpu.make_async_copy(pos_hbm, pos_v, semp)
    idx_cp.start()
    pos_cp.start()
    idx_cp.wait()

    evecs = [lax.iota(jnp.int32, 16) + 16 * ec for ec in range(4)]
    ehis = [(lax.iota(jnp.int32, 16) + 16 * ec) // 8 for ec in range(4)]
    elos = [(lax.iota(jnp.int32, 16) + 16 * ec) % 8 for ec in range(4)]

    def fire_gather(s, b):
        pltpu.async_copy(
            tok_hbm.at[idx_v.at[s // 8, s % 8]], gath_v.at[b], semg[b])

    for b in range(_NBUF):
        fire_gather(b, b)
    pos_cp.wait()

    def step_body(it, carry):
        for b in range(_NBUF):
            s = it * _NBUF + b
            pltpu.make_async_copy(
                tok_hbm.at[pl.ds(0, _BTILE)], gath_v.at[b], semg[b]).wait()
            pl.when(it > 0)(
                lambda: pltpu.make_async_copy(
                    tok_hbm.at[pl.ds(0, _BTILE)], gath_v.at[b],
                    semo[b]).wait())

            g = gath_v.at[b]
            bufv = jnp.full((16,), b, jnp.int32)
            pcs = [pos_v[s, pl.ds(ec * _LANES, _LANES)] for ec in range(4)]

            @plsc.parallel_loop(0, _BTILE, 1, unroll=16)
            def add_t(bb):
                bv = jnp.full((16,), bb, jnp.int32)
                for ec in range(4):
                    vals = g[bb, pl.ds(ec * _LANES, _LANES)] + pcs[ec]
                    plsc.store_scatter(outb_v, [bufv, ehis[ec], elos[ec], bv],
                                       vals)

            pl.when(it < (_SEQ // _NBUF) - 1)(lambda: fire_gather(s + _NBUF, b))
            pltpu.async_copy(
                outb_v.at[b, :, :, pl.ds(0, _BTILE)],
                out_hbm.at[s, pl.ds(0, 8), wid], semo[b])
        return carry

    lax.fori_loop(0, _SEQ // _NBUF, step_body, 0)
    for b in range(_NBUF):
        pltpu.make_async_copy(
            tok_hbm.at[pl.ds(0, _BTILE)], gath_v.at[b], semo[b]).wait()


def kernel(x, token_table, pos_table):
    x32 = x.astype(jnp.int32)
    x4 = x32.T.reshape(_STILE, 8, _NW, _BTILE).transpose(0, 2, 1, 3)
    f = pl.kernel(
        _sc_body,
        mesh=plsc.VectorSubcoreMesh(core_axis_name="c", subcore_axis_name="s"),
        compiler_params=pltpu.CompilerParams(
            use_tc_tiling_on_sc=False, needs_layout_passes=False),
        out_type=jax.ShapeDtypeStruct((_SEQ, 8, _NW, 8, _BTILE), jnp.float32),
        scratch_types=[
            pltpu.VMEM((_STILE, 8, _BTILE), jnp.int32),
            pltpu.VMEM((_NBUF, _BTILE, _EMBED), jnp.float32),
            pltpu.VMEM((_NBUF, 8, 8, _BTILE + 1), jnp.float32),
            pltpu.VMEM((_SEQ, _EMBED), jnp.float32),
        ] + [pltpu.SemaphoreType.DMA] * (2 * _NBUF + 2),
    )
    y5 = f(x4, token_table, pos_table)
    return y5.transpose(2, 4, 0, 1, 3).reshape(_BATCH, _SEQ, _EMBED)
